# Initial kernel scaffold; baseline (speedup 1.0000x reference)
import jax
import jax.numpy as jnp
from jax.experimental import pallas as pl


def kernel(x, kernel):
    raise NotImplementedError("write your pallas kernel here")


# speedup vs baseline: 487.3139x; 487.3139x over previous
import jax
import jax.numpy as jnp
from jax.experimental import pallas as pl
from jax.experimental.pallas import tpu as pltpu

_K = 256
_N = 4096
_BIG = 1e30


def _dtw_body(xp_ref, k_ref, out_ref):
    kern = k_ref[...]
    lane = jax.lax.broadcasted_iota(jnp.int32, (1, _K), 1)
    first = lane == 0

    def step(d, carry):
        a1, shp, w = carry
        xd = xp_ref[d, 0]
        w = jnp.where(first, xd, pltpu.roll(w, 1, 1))
        c = kern - w
        c = c * c
        sh1 = jnp.where(first, 0.0, pltpu.roll(a1, 1, 1))
        anew = c + jnp.minimum(jnp.minimum(a1, shp), sh1)

        @pl.when(d >= _K - 1)
        def _():
            out_ref[pl.ds(d - (_K - 1), 1), :] = anew[:, _K - 1:]

        return anew, sh1, w

    a0 = jnp.full((1, _K), _BIG, jnp.float32)
    shp0 = jnp.where(first, 0.0, a0)
    w0 = jnp.zeros((1, _K), jnp.float32)
    jax.lax.fori_loop(0, _N + _K - 1, step, (a0, shp0, w0))


def _run(x, kern, interpret=False):
    xp = jnp.concatenate([x, jnp.zeros((_K,), jnp.float32)]).reshape(_N + _K, 1)
    k2 = kern.reshape(1, _K)
    out = pl.pallas_call(
        _dtw_body,
        out_shape=jax.ShapeDtypeStruct((_N, 1), jnp.float32),
        interpret=interpret,
    )(xp, k2)
    return out.reshape(_N)


def kernel(x, kernel):
    return _run(x, kernel)


# speedup vs baseline: 890.1446x; 1.8266x over previous
import jax
import jax.numpy as jnp
from jax.experimental import pallas as pl
from jax.experimental.pallas import tpu as pltpu

_K = 256
_N = 4096
_BIG = 1e30
_NBLK = (_N + _K) // _K


def _dtw_body(xp_ref, k_ref, out_ref):
    kern = k_ref[...]
    first = jax.lax.broadcasted_iota(jnp.int32, (1, _K), 1) == 0

    def inner(t, c2):
        a1, shp, w, cap, sup = c2
        w = jnp.where(first, sup, pltpu.roll(w, 1, 1))
        sup = pltpu.roll(sup, _K - 1, 1)
        c = kern - w
        c = c * c
        r1 = pltpu.roll(a1, 1, 1)
        sh1 = jnp.where(first, 0.0, r1)
        cap = jnp.where(first, r1, pltpu.roll(cap, 1, 1))
        anew = c + jnp.minimum(jnp.minimum(a1, shp), sh1)
        return (anew, sh1, w, cap, sup)

    def outer(b, carry):
        a1, shp, w, cap = carry
        sup = xp_ref[pl.ds(b, 1), :]
        a1, shp, w, cap, _ = jax.lax.fori_loop(
            0, _K, inner, (a1, shp, w, cap, sup))

        @pl.when(b >= 1)
        def _():
            out_ref[pl.ds(b - 1, 1), :] = cap

        return (a1, shp, w, cap)

    a0 = jnp.full((1, _K), _BIG, jnp.float32)
    shp0 = jnp.where(first, 0.0, a0)
    zeros = jnp.zeros((1, _K), jnp.float32)
    jax.lax.fori_loop(0, _NBLK, outer, (a0, shp0, zeros, zeros))


def _run(x, kern, interpret=False):
    xp = jnp.concatenate(
        [x, jnp.zeros((_NBLK * _K - _N,), jnp.float32)]).reshape(_NBLK, _K)
    k2 = kern.reshape(1, _K)
    out = pl.pallas_call(
        _dtw_body,
        out_shape=jax.ShapeDtypeStruct((_NBLK - 1, _K), jnp.float32),
        interpret=interpret,
    )(xp, k2)
    return jnp.flip(out, axis=1).reshape(_N)


def kernel(x, kernel):
    return _run(x, kernel)
